# baseline (speedup 1.0000x reference)
import functools

import jax
import jax.numpy as jnp
from jax import lax
from jax.experimental import pallas as pl
from jax.experimental.pallas import tpu as pltpu
from jax.experimental.pallas import tpu_sc as plsc


def kernel(batch, weight):
    B, = batch.shape
    V, D = weight.shape
    NC, NS, L = 2, 16, 16
    NW = NC * NS
    SUP = 256
    NBUF = 4
    NSLOT = 4
    TAIL = V - ((V // SUP) * SUP)
    VMAIN = V - TAIL
    SUP_PER_W = (VMAIN // SUP) // NW
    SUP_REM = (VMAIN // SUP) % NW
    assert B % L == 0 and D % L == 0

    wt = weight.T
    mesh = plsc.VectorSubcoreMesh(core_axis_name="c", subcore_axis_name="s")
    OUTR = B + L

    @functools.partial(
        pl.kernel,
        mesh=mesh,
        compiler_params=pltpu.CompilerParams(needs_layout_passes=False),
        out_type=jax.ShapeDtypeStruct((OUTR, 2 * D), jnp.float32),
        scratch_types=[
            pltpu.VMEM((B,), jnp.int32),
            pltpu.VMEM((B,), jnp.int32),
            pltpu.VMEM((NBUF, D, SUP), jnp.float32),
            pltpu.VMEM((D, TAIL), jnp.float32),
            pltpu.VMEM((NSLOT * L, 2 * D), jnp.float32),
            pltpu.SemaphoreType.DMA((NBUF,)),
            pltpu.SemaphoreType.DMA((NSLOT,)),
        ],
    )
    def _emb(idx_hbm, table_hbm, out_hbm, idx_v, mp,
             stg, stgT, rowb, sem_in, sem_out):
        smp = idx_v
        wid = lax.axis_index("s") * NC + lax.axis_index("c")
        is_last = wid == (NW - 1)
        lo = wid * (SUP_PER_W * SUP)
        nsup = lax.select(is_last, SUP_PER_W + SUP_REM, SUP_PER_W)
        hi = lax.select(is_last, V, lo + SUP_PER_W * SUP)
        iota = lax.iota(jnp.int32, L)

        def issue(s):
            off = pl.multiple_of(lo + s * SUP, SUP)
            b = s % NBUF
            for r in range(D // 8):
                pltpu.async_copy(
                    table_hbm.at[pl.ds(r * 8, 8), pl.ds(off, SUP)],
                    stg.at[b].at[pl.ds(r * 8, 8)],
                    sem_in.at[b],
                )

        for s0 in range(NBUF - 1):
            issue(s0)

        pltpu.sync_copy(idx_hbm, idx_v)

        def scan_body(g, cnt):
            v = idx_v[pl.ds(g * L, L)]
            m = (v >= lo) & (v < hi)
            p = plsc.all_reduce_population_count(m)[0]

            @pl.when(p > 0)
            def _():
                pos = plsc.cumsum(m.astype(jnp.int32)) - 1 + cnt
                packed = ((iota + g * L) << 15) | (v - lo)
                plsc.store_scatter(mp, [pos], packed, mask=m)

            return cnt + p

        cnt = lax.fori_loop(0, B // L, scan_body, 0)
        n_mc = (cnt + L - 1) // L

        def process_super(buf, rlo, swidth, ns):
            def resc(c, scnt):
                vp = mp[pl.ds(c * L, L)]
                vi = vp & 0x7FFF
                m = (vi >= rlo) & (vi < rlo + swidth) & ((iota + c * L) < cnt)
                p = plsc.all_reduce_population_count(m)[0]

                @pl.when(p > 0)
                def _():
                    pos = plsc.cumsum(m.astype(jnp.int32)) - 1 + scnt
                    packed = (((vp >> 15) & 0x3FFF) << 9) | (vi - rlo)
                    plsc.store_scatter(smp, [pos], packed, mask=m)

                return scnt + p

            scnt = lax.fori_loop(0, n_mc, resc, 0)

            def chunk(c, ns):
                vp = smp[pl.ds(c * L, L)]
                valid = (iota + c * L) < scnt
                il = jnp.where(valid, vp & 0x1FF, 0)
                jv = jnp.where(valid, (vp >> 9) & 0x3FFF, B + iota)
                k = ns % NSLOT

                @pl.when(ns >= NSLOT)
                def _():
                    pltpu.make_async_copy(
                        rowb.at[pl.ds(0, L)], out_hbm.at[B + iota],
                        sem_out.at[k],
                    ).wait()

                slot = rowb.at[pl.ds(k * L, L)]
                for d in range(D):
                    dv = jnp.full((L,), d, jnp.int32)
                    x = plsc.load_gather(buf, [dv, il])
                    plsc.store_scatter(slot, [iota, dv], x)
                pltpu.async_copy(slot, out_hbm.at[jv], sem_out.at[k])
                return ns + 1

            return lax.fori_loop(0, (scnt + L - 1) // L, chunk, ns)

        def sup_body(s, ns):
            @pl.when(s + (NBUF - 1) < nsup)
            def _():
                issue(s + (NBUF - 1))

            b = s % NBUF
            pltpu.make_async_copy(
                table_hbm.at[:, pl.ds(0, SUP)], stg.at[b], sem_in.at[b]
            ).wait()
            return process_super(stg.at[b], s * SUP, SUP, ns)

        ns = lax.fori_loop(0, nsup, sup_body, 0)

        pltpu.sync_copy(table_hbm.at[:, pl.ds(VMAIN, TAIL)], stgT)
        ns = process_super(stgT, VMAIN - lo, TAIL, ns)

        for k in range(NSLOT):
            @pl.when(ns > k)
            def _():
                pltpu.make_async_copy(
                    rowb.at[pl.ds(0, L)], out_hbm.at[B + iota], sem_out.at[k]
                ).wait()

    return _emb(batch, wt)[:B, :D]


# speedup vs baseline: 1.4711x; 1.4711x over previous
import functools

import jax
import jax.numpy as jnp
from jax import lax
from jax.experimental import pallas as pl
from jax.experimental.pallas import tpu as pltpu
from jax.experimental.pallas import tpu_sc as plsc


def kernel(batch, weight):
    B, = batch.shape
    V, D = weight.shape
    NC, NS, L = 2, 16, 16
    NW = NC * NS
    SUP = 512
    NBUF = 2
    NSLOT = 4
    TAIL = V - ((V // SUP) * SUP)
    VMAIN = V - TAIL
    SUP_PER_W = (VMAIN // SUP) // NW
    SUP_REM = (VMAIN // SUP) % NW
    assert B % L == 0 and D % L == 0

    wt = weight.T
    mesh = plsc.VectorSubcoreMesh(core_axis_name="c", subcore_axis_name="s")
    OUTR = B + L

    @functools.partial(
        pl.kernel,
        mesh=mesh,
        compiler_params=pltpu.CompilerParams(needs_layout_passes=False),
        out_type=jax.ShapeDtypeStruct((OUTR, 2 * D), jnp.float32),
        scratch_types=[
            pltpu.VMEM((B,), jnp.int32),
            pltpu.VMEM((B,), jnp.int32),
            pltpu.VMEM((NBUF, D, SUP), jnp.float32),
            pltpu.VMEM((D, TAIL), jnp.float32),
            pltpu.VMEM((NSLOT * L, 2 * D), jnp.float32),
            pltpu.SemaphoreType.DMA((NBUF,)),
            pltpu.SemaphoreType.DMA((NSLOT,)),
        ],
    )
    def _emb(idx_hbm, table_hbm, out_hbm, idx_v, mp,
             stg, stgT, rowb, sem_in, sem_out):
        smp = idx_v
        wid = lax.axis_index("s") * NC + lax.axis_index("c")
        is_last = wid == (NW - 1)
        lo = wid * (SUP_PER_W * SUP)
        nsup = lax.select(is_last, SUP_PER_W + SUP_REM, SUP_PER_W)
        hi = lax.select(is_last, V, lo + SUP_PER_W * SUP)
        iota = lax.iota(jnp.int32, L)

        def issue(s):
            off = pl.multiple_of(lo + s * SUP, SUP)
            b = s % NBUF
            for r in range(D // 8):
                pltpu.async_copy(
                    table_hbm.at[pl.ds(r * 8, 8), pl.ds(off, SUP)],
                    stg.at[b].at[pl.ds(r * 8, 8)],
                    sem_in.at[b],
                )

        for s0 in range(NBUF - 1):
            issue(s0)

        pltpu.sync_copy(idx_hbm, idx_v)

        def scan_body(g, cnt_v):
            v = idx_v[pl.ds(g * L, L)]
            m = (v >= lo) & (v < hi)
            pos = plsc.cumsum(m.astype(jnp.int32)) - 1 + cnt_v
            packed = ((iota + g * L) << 15) | (v - lo)
            plsc.store_scatter(mp, [pos], packed, mask=m)
            return cnt_v + plsc.all_reduce_population_count(m)

        cnt_v = lax.fori_loop(0, B // L, scan_body,
                              jnp.zeros((L,), jnp.int32), unroll=4)
        cnt = cnt_v[0]
        n_mc = (cnt + L - 1) // L

        def process_super(buf, rlo, swidth, ns):
            def resc(c, scnt_v):
                vp = mp[pl.ds(c * L, L)]
                vi = vp & 0x7FFF
                m = (vi >= rlo) & (vi < rlo + swidth) & ((iota + c * L) < cnt)
                pos = plsc.cumsum(m.astype(jnp.int32)) - 1 + scnt_v
                packed = (((vp >> 15) & 0x3FFF) << 9) | (vi - rlo)
                plsc.store_scatter(smp, [pos], packed, mask=m)
                return scnt_v + plsc.all_reduce_population_count(m)

            scnt = lax.fori_loop(0, n_mc, resc,
                                 jnp.zeros((L,), jnp.int32))[0]

            def chunk(c, ns):
                vp = smp[pl.ds(c * L, L)]
                valid = (iota + c * L) < scnt
                il = jnp.where(valid, vp & 0x1FF, 0)
                jv = jnp.where(valid, (vp >> 9) & 0x3FFF, B + iota)
                k = ns % NSLOT

                @pl.when(ns >= NSLOT)
                def _():
                    pltpu.make_async_copy(
                        rowb.at[pl.ds(0, L)], out_hbm.at[B + iota],
                        sem_out.at[k],
                    ).wait()

                slot = rowb.at[pl.ds(k * L, L)]
                for d in range(D):
                    dv = jnp.full((L,), d, jnp.int32)
                    x = plsc.load_gather(buf, [dv, il])
                    plsc.store_scatter(slot, [iota, dv], x)
                pltpu.async_copy(slot, out_hbm.at[jv], sem_out.at[k])
                return ns + 1

            return lax.fori_loop(0, (scnt + L - 1) // L, chunk, ns)

        def sup_body(s, ns):
            @pl.when(s + (NBUF - 1) < nsup)
            def _():
                issue(s + (NBUF - 1))

            b = s % NBUF
            pltpu.make_async_copy(
                table_hbm.at[:, pl.ds(0, SUP)], stg.at[b], sem_in.at[b]
            ).wait()
            return process_super(stg.at[b], s * SUP, SUP, ns)

        ns = lax.fori_loop(0, nsup, sup_body, 0)

        pltpu.sync_copy(table_hbm.at[:, pl.ds(VMAIN, TAIL)], stgT)
        ns = process_super(stgT, VMAIN - lo, TAIL, ns)

        for k in range(NSLOT):
            @pl.when(ns > k)
            def _():
                pltpu.make_async_copy(
                    rowb.at[pl.ds(0, L)], out_hbm.at[B + iota], sem_out.at[k]
                ).wait()

    return _emb(batch, wt)[:B, :D]
